# baseline (speedup 1.0000x reference)
import functools

import jax
import jax.numpy as jnp
from jax import lax
from jax.experimental import pallas as pl
from jax.experimental.pallas import tpu as pltpu
from jax.experimental.pallas import tpu_sc as plsc

_NUM_WORKERS = 32
_CHUNK = 400
_NBUF = 4

_WBLK = 32768
_OBLK = 4096


@functools.cache
def _build_gather(n_rows, table_rows, dim, chunk):
    n_per_w = n_rows // _NUM_WORKERS
    n_chunks = n_per_w // chunk
    n_steady = n_chunks - _NBUF
    assert n_steady % _NBUF == 0 and n_steady >= 0
    mesh = plsc.VectorSubcoreMesh(core_axis_name="c", subcore_axis_name="s")

    @functools.partial(
        pl.kernel,
        mesh=mesh,
        out_type=jax.ShapeDtypeStruct((n_rows, dim), jnp.float32),
        scratch_types=[
            pltpu.VMEM((n_per_w,), jnp.int32),
            pltpu.VMEM((_NBUF, chunk, dim), jnp.float32),
            [pltpu.SemaphoreType.DMA] * _NBUF,
            [pltpu.SemaphoreType.DMA] * _NBUF,
        ],
        compiler_params=pltpu.CompilerParams(use_tc_tiling_on_sc=False),
    )
    def emb(x_hbm, w_hbm, out_hbm, idx_v, rows_v, sem_g, sem_s):
        wid = lax.axis_index("s") * 2 + lax.axis_index("c")
        base = wid * n_per_w

        pltpu.sync_copy(x_hbm.at[pl.ds(base, n_per_w)], idx_v)

        for b in range(_NBUF):
            pltpu.async_copy(
                w_hbm.at[idx_v.at[pl.ds(b * chunk, chunk)]],
                rows_v.at[b],
                sem_g[b],
            )

        def body(p, carry):
            for b in range(_NBUF):
                c = p * _NBUF + b
                off = base + c * chunk
                pltpu.make_async_copy(
                    w_hbm.at[idx_v.at[pl.ds(c * chunk, chunk)]],
                    rows_v.at[b],
                    sem_g[b],
                ).wait()
                pltpu.async_copy(
                    rows_v.at[b],
                    out_hbm.at[pl.ds(off, chunk)],
                    sem_s[b],
                )
                pltpu.make_async_copy(
                    rows_v.at[b],
                    out_hbm.at[pl.ds(off, chunk)],
                    sem_s[b],
                ).wait()
                pltpu.async_copy(
                    w_hbm.at[idx_v.at[pl.ds((c + _NBUF) * chunk, chunk)]],
                    rows_v.at[b],
                    sem_g[b],
                )
            return carry

        lax.fori_loop(0, n_steady // _NBUF, body, 0)

        for b in range(_NBUF):
            c = n_steady + b
            off = base + c * chunk
            pltpu.make_async_copy(
                w_hbm.at[idx_v.at[pl.ds(c * chunk, chunk)]],
                rows_v.at[b],
                sem_g[b],
            ).wait()
            pltpu.async_copy(
                rows_v.at[b],
                out_hbm.at[pl.ds(off, chunk)],
                sem_s[b],
            )
        for b in range(_NBUF):
            c = n_steady + b
            off = base + c * chunk
            pltpu.make_async_copy(
                rows_v.at[b],
                out_hbm.at[pl.ds(off, chunk)],
                sem_s[b],
            ).wait()

    return emb


def _w_transpose_body(wt_ref, o_ref):
    xb = wt_ref[...]
    o_ref[:, 0 : xb.shape[0]] = xb.T


def _o_transpose_body(i_ref, o_ref):
    dim = o_ref.shape[1]
    tr = i_ref[...].T
    o_ref[0] = tr[0:dim, :]
    o_ref[1] = tr[dim : 2 * dim, :]


def kernel(x, weight):
    b, s = x.shape
    vocab, dim = weight.shape
    n_rows = b * s

    wt = weight.T
    wpad = pl.pallas_call(
        _w_transpose_body,
        grid=(-(-vocab // _WBLK),),
        in_specs=[pl.BlockSpec((dim, _WBLK), lambda i: (0, i))],
        out_specs=pl.BlockSpec((_WBLK, 2 * dim), lambda i: (i, 0)),
        out_shape=jax.ShapeDtypeStruct((vocab, 2 * dim), jnp.float32),
    )(wt)
    wlin = wpad.reshape(2 * vocab, dim)

    xf = (
        x.reshape(b, s // 2, 2).transpose(1, 0, 2).reshape(-1).astype(jnp.int32)
        * 2
    )
    out_lin = _build_gather(n_rows, 2 * vocab, dim, _CHUNK)(xf, wlin)

    o2 = out_lin.reshape(n_rows // 2, 2 * dim)
    nt = b // _OBLK
    out_phys = pl.pallas_call(
        _o_transpose_body,
        grid=(s // 2, nt),
        in_specs=[
            pl.BlockSpec((_OBLK, 2 * dim), lambda J, t: (J * nt + t, 0))
        ],
        out_specs=pl.BlockSpec((2, dim, _OBLK), lambda J, t: (J, 0, t)),
        out_shape=jax.ShapeDtypeStruct((s, dim, b), jnp.float32),
    )(o2)
    return jnp.transpose(out_phys, (2, 0, 1))


# speedup vs baseline: 1.0358x; 1.0358x over previous
import functools

import jax
import jax.numpy as jnp
from jax import lax
from jax.experimental import pallas as pl
from jax.experimental.pallas import tpu as pltpu
from jax.experimental.pallas import tpu_sc as plsc

_NUM_WORKERS = 32
_CHUNK = 400
_NBUF = 4

_WBLK = 32768
_OBLK = 16384


@functools.cache
def _build_gather(n_rows, table_rows, dim, chunk):
    n_per_w = n_rows // _NUM_WORKERS
    n_chunks = n_per_w // chunk
    n_steady = n_chunks - _NBUF
    assert n_steady % _NBUF == 0 and n_steady >= 0
    mesh = plsc.VectorSubcoreMesh(core_axis_name="c", subcore_axis_name="s")

    @functools.partial(
        pl.kernel,
        mesh=mesh,
        out_type=jax.ShapeDtypeStruct((n_rows, dim), jnp.float32),
        scratch_types=[
            pltpu.VMEM((n_per_w,), jnp.int32),
            pltpu.VMEM((_NBUF, chunk, dim), jnp.float32),
            [pltpu.SemaphoreType.DMA] * _NBUF,
            [pltpu.SemaphoreType.DMA] * _NBUF,
        ],
        compiler_params=pltpu.CompilerParams(use_tc_tiling_on_sc=False),
    )
    def emb(x_hbm, w_hbm, out_hbm, idx_v, rows_v, sem_g, sem_s):
        wid = lax.axis_index("s") * 2 + lax.axis_index("c")
        base = wid * n_per_w

        pltpu.sync_copy(x_hbm.at[pl.ds(base, n_per_w)], idx_v)

        for b in range(_NBUF):
            pltpu.async_copy(
                w_hbm.at[idx_v.at[pl.ds(b * chunk, chunk)]],
                rows_v.at[b],
                sem_g[b],
            )

        def body(p, carry):
            for b in range(_NBUF):
                c = p * _NBUF + b
                off = base + c * chunk
                pltpu.make_async_copy(
                    w_hbm.at[idx_v.at[pl.ds(c * chunk, chunk)]],
                    rows_v.at[b],
                    sem_g[b],
                ).wait()
                pltpu.async_copy(
                    rows_v.at[b],
                    out_hbm.at[pl.ds(off, chunk)],
                    sem_s[b],
                )
                pltpu.make_async_copy(
                    rows_v.at[b],
                    out_hbm.at[pl.ds(off, chunk)],
                    sem_s[b],
                ).wait()
                pltpu.async_copy(
                    w_hbm.at[idx_v.at[pl.ds((c + _NBUF) * chunk, chunk)]],
                    rows_v.at[b],
                    sem_g[b],
                )
            return carry

        lax.fori_loop(0, n_steady // _NBUF, body, 0)

        for b in range(_NBUF):
            c = n_steady + b
            off = base + c * chunk
            pltpu.make_async_copy(
                w_hbm.at[idx_v.at[pl.ds(c * chunk, chunk)]],
                rows_v.at[b],
                sem_g[b],
            ).wait()
            pltpu.async_copy(
                rows_v.at[b],
                out_hbm.at[pl.ds(off, chunk)],
                sem_s[b],
            )
        for b in range(_NBUF):
            c = n_steady + b
            off = base + c * chunk
            pltpu.make_async_copy(
                rows_v.at[b],
                out_hbm.at[pl.ds(off, chunk)],
                sem_s[b],
            ).wait()

    return emb


def _w_transpose_body(wt_ref, o_ref):
    xb = wt_ref[...]
    o_ref[:, 0 : xb.shape[0]] = xb.T


def _o_transpose_body(i_ref, o_ref):
    dim = o_ref.shape[1]
    tr = i_ref[...].T
    o_ref[0] = tr[0:dim, :]
    o_ref[1] = tr[dim : 2 * dim, :]


def kernel(x, weight):
    b, s = x.shape
    vocab, dim = weight.shape
    n_rows = b * s

    wt = weight.T
    wpad = pl.pallas_call(
        _w_transpose_body,
        grid=(-(-vocab // _WBLK),),
        in_specs=[pl.BlockSpec((dim, _WBLK), lambda i: (0, i))],
        out_specs=pl.BlockSpec((_WBLK, 2 * dim), lambda i: (i, 0)),
        out_shape=jax.ShapeDtypeStruct((vocab, 2 * dim), jnp.float32),
    )(wt)
    wlin = wpad.reshape(2 * vocab, dim)

    xf = (
        x.reshape(b, s // 2, 2).transpose(1, 0, 2).reshape(-1).astype(jnp.int32)
        * 2
    )
    out_lin = _build_gather(n_rows, 2 * vocab, dim, _CHUNK)(xf, wlin)

    o2 = out_lin.reshape(n_rows // 2, 2 * dim)
    nt = b // _OBLK
    out_phys = pl.pallas_call(
        _o_transpose_body,
        grid=(s // 2, nt),
        in_specs=[
            pl.BlockSpec((_OBLK, 2 * dim), lambda J, t: (J * nt + t, 0))
        ],
        out_specs=pl.BlockSpec((2, dim, _OBLK), lambda J, t: (J, 0, t)),
        out_shape=jax.ShapeDtypeStruct((s, dim, b), jnp.float32),
    )(o2)
    return jnp.transpose(out_phys, (2, 0, 1))


# speedup vs baseline: 1.3250x; 1.2792x over previous
import functools

import jax
import jax.numpy as jnp
from jax import lax
from jax.experimental import pallas as pl
from jax.experimental.pallas import tpu as pltpu
from jax.experimental.pallas import tpu_sc as plsc

_NUM_WORKERS = 32
_CHUNK = 400
_NBUF = 4

_WBLK = 32768
_OBLK = 16384


@functools.cache
def _build_gather(n_rows, table_rows, dim, chunk):
    n_per_w = n_rows // _NUM_WORKERS
    n_chunks = n_per_w // chunk
    n_steady = n_chunks - _NBUF
    assert n_steady % _NBUF == 0 and n_steady >= 0
    mesh = plsc.VectorSubcoreMesh(core_axis_name="c", subcore_axis_name="s")

    @functools.partial(
        pl.kernel,
        mesh=mesh,
        out_type=jax.ShapeDtypeStruct((n_rows, 2 * dim), jnp.float32),
        scratch_types=[
            pltpu.VMEM((n_per_w,), jnp.int32),
            pltpu.VMEM((_NBUF, chunk, dim), jnp.float32),
            [pltpu.SemaphoreType.DMA] * _NBUF,
            [pltpu.SemaphoreType.DMA] * _NBUF,
        ],
        compiler_params=pltpu.CompilerParams(use_tc_tiling_on_sc=False),
    )
    def emb(x_hbm, w_hbm, out_hbm, idx_v, rows_v, sem_g, sem_s):
        wid = lax.axis_index("s") * 2 + lax.axis_index("c")
        base = wid * n_per_w

        pltpu.sync_copy(x_hbm.at[pl.ds(base, n_per_w)], idx_v)

        for b in range(_NBUF):
            pltpu.async_copy(
                w_hbm.at[idx_v.at[pl.ds(b * chunk, chunk)]],
                rows_v.at[b],
                sem_g[b],
            )

        def body(p, carry):
            for b in range(_NBUF):
                c = p * _NBUF + b
                off = base + c * chunk
                pltpu.make_async_copy(
                    w_hbm.at[idx_v.at[pl.ds(c * chunk, chunk)]],
                    rows_v.at[b],
                    sem_g[b],
                ).wait()
                pltpu.async_copy(
                    rows_v.at[b],
                    out_hbm.at[pl.ds(off, chunk), pl.ds(0, dim)],
                    sem_s[b],
                )
                pltpu.make_async_copy(
                    rows_v.at[b],
                    out_hbm.at[pl.ds(off, chunk), pl.ds(0, dim)],
                    sem_s[b],
                ).wait()
                pltpu.async_copy(
                    w_hbm.at[idx_v.at[pl.ds((c + _NBUF) * chunk, chunk)]],
                    rows_v.at[b],
                    sem_g[b],
                )
            return carry

        lax.fori_loop(0, n_steady // _NBUF, body, 0)

        for b in range(_NBUF):
            c = n_steady + b
            off = base + c * chunk
            pltpu.make_async_copy(
                w_hbm.at[idx_v.at[pl.ds(c * chunk, chunk)]],
                rows_v.at[b],
                sem_g[b],
            ).wait()
            pltpu.async_copy(
                rows_v.at[b],
                out_hbm.at[pl.ds(off, chunk), pl.ds(0, dim)],
                sem_s[b],
            )
        for b in range(_NBUF):
            c = n_steady + b
            off = base + c * chunk
            pltpu.make_async_copy(
                rows_v.at[b],
                out_hbm.at[pl.ds(off, chunk), pl.ds(0, dim)],
                sem_s[b],
            ).wait()

    return emb


def _w_transpose_body(wt_ref, o_ref):
    xb = wt_ref[...]
    o_ref[:, 0 : xb.shape[0]] = xb.T


def _o_transpose_body(i_ref, o_ref):
    dim = o_ref.shape[0]
    o_ref[...] = i_ref[...].T[0:dim, :]


def kernel(x, weight):
    b, s = x.shape
    vocab, dim = weight.shape
    n_rows = b * s

    wt = weight.T
    wpad = pl.pallas_call(
        _w_transpose_body,
        grid=(-(-vocab // _WBLK),),
        in_specs=[pl.BlockSpec((dim, _WBLK), lambda i: (0, i))],
        out_specs=pl.BlockSpec((_WBLK, 2 * dim), lambda i: (i, 0)),
        out_shape=jax.ShapeDtypeStruct((vocab, 2 * dim), jnp.float32),
    )(wt)
    wlin = wpad.reshape(2 * vocab, dim)

    xf = x.T.reshape(-1).astype(jnp.int32) * 2
    out_pad = _build_gather(n_rows, 2 * vocab, dim, _CHUNK)(xf, wlin)

    nt = b // _OBLK
    out_phys = pl.pallas_call(
        _o_transpose_body,
        grid=(s, nt),
        in_specs=[
            pl.BlockSpec((_OBLK, 2 * dim), lambda j, t: (j * nt + t, 0))
        ],
        out_specs=pl.BlockSpec((None, dim, _OBLK), lambda j, t: (j, 0, t)),
        out_shape=jax.ShapeDtypeStruct((s, dim, b), jnp.float32),
    )(out_pad)
    return jnp.transpose(out_phys, (2, 0, 1))
